# baseline (speedup 1.0000x reference)
import functools

import jax
import jax.numpy as jnp
from jax import lax
from jax.experimental import pallas as pl

N = 100000
TOP_N = 10
BLK = 1024
NBLK = (N + BLK - 1) // BLK
NPAD = NBLK * BLK
NEG = -1e30


def _sims_body(x_ref, q_ref, o_ref, *, d):
    x = x_ref[...]
    q = q_ref[...]
    xx = x * x
    dot = lax.dot_general(q, x, (((1,), (1,)), ((), ())),
                          preferred_element_type=jnp.float32,
                          precision=lax.Precision.HIGHEST)
    ones = jnp.ones((1, d), jnp.float32)
    sq = lax.dot_general(ones, xx, (((1,), (1,)), ((), ())),
                         preferred_element_type=jnp.float32,
                         precision=lax.Precision.HIGHEST)
    qn = jnp.maximum(jnp.sqrt(jnp.sum(q * q)), 1e-8)
    xn = jnp.maximum(jnp.sqrt(sq), 1e-8)
    sims = dot / (xn * qn)
    i = pl.program_id(0)
    row = i * BLK + lax.broadcasted_iota(jnp.int32, (1, BLK), 1)
    o_ref[...] = jnp.where(row < N, sims, NEG).reshape(1, 1, BLK)


def _sims(x, q):
    d = x.shape[1]
    return pl.pallas_call(
        functools.partial(_sims_body, d=d),
        grid=(NBLK,),
        in_specs=[
            pl.BlockSpec((BLK, d), lambda i: (i, 0)),
            pl.BlockSpec((1, d), lambda i: (0, 0)),
        ],
        out_specs=pl.BlockSpec((1, 1, BLK), lambda i: (i, 0, 0)),
        out_shape=jax.ShapeDtypeStruct((NBLK, 1, BLK), jnp.float32),
    )(x, q).reshape(NBLK, BLK)


def _topk_body(s_ref, vals_ref, idx_ref):
    s = s_ref[...]
    denom = jnp.sum(jnp.exp(s))
    lin = (lax.broadcasted_iota(jnp.int32, (NBLK, BLK), 0) * BLK
           + lax.broadcasted_iota(jnp.int32, (NBLK, BLK), 1))
    lane = lax.broadcasted_iota(jnp.int32, (1, 16), 1)
    vals = jnp.full((1, 16), NEG, jnp.float32)
    idxs = jnp.zeros((1, 16), jnp.int32)
    for k in range(TOP_N):
        m = jnp.max(s)
        sel = s == m
        cand = jnp.min(jnp.where(sel, lin, jnp.int32(2 ** 30)))
        vals = jnp.where(lane == k, m, vals)
        idxs = jnp.where(lane == k, cand, idxs)
        s = jnp.where(lin == cand, NEG, s)
    vals_ref[...] = jnp.exp(vals) / denom
    idx_ref[...] = idxs


def _topk(sims):
    vals, idxs = pl.pallas_call(
        _topk_body,
        out_shape=(jax.ShapeDtypeStruct((1, 16), jnp.float32),
                   jax.ShapeDtypeStruct((1, 16), jnp.int32)),
    )(sims)
    return vals[0, :TOP_N], idxs[0, :TOP_N]


def kernel(gemini_sections, gemini_chapters, gemini_pages,
           voyager_sections, voyager_chapters, voyager_pages,
           gemini_query_embedding, voyager_query_embedding):
    out = []
    for emb, q in ((gemini_sections, gemini_query_embedding),
                   (gemini_chapters, gemini_query_embedding),
                   (gemini_pages, gemini_query_embedding),
                   (voyager_sections, voyager_query_embedding),
                   (voyager_chapters, voyager_query_embedding),
                   (voyager_pages, voyager_query_embedding)):
        vals, idxs = _topk(_sims(emb, q))
        out.append(vals)
        out.append(idxs)
    return tuple(out)


# speedup vs baseline: 3.8756x; 3.8756x over previous
import functools

import jax
import jax.numpy as jnp
from jax import lax
from jax.experimental import pallas as pl

N = 100000
TOP_N = 10
BLK = 1024
NBLK = (N + BLK - 1) // BLK
NPAD = NBLK * BLK
NEG = -1e30


def _sims_body(x_ref, q_ref, o_ref, *, d):
    x = x_ref[...]
    q = q_ref[...]
    dot = jnp.sum(x * q, axis=1)
    sq = jnp.sum(x * x, axis=1)
    qn = jnp.maximum(jnp.sqrt(jnp.sum(q * q)), 1e-8)
    xn = jnp.maximum(jnp.sqrt(sq), 1e-8)
    sims = dot / (xn * qn)
    i = pl.program_id(0)
    row = i * BLK + lax.broadcasted_iota(jnp.int32, (1, BLK), 1)
    o_ref[...] = jnp.where(row < N, sims.reshape(1, BLK), NEG).reshape(1, 1, BLK)


def _sims(x, q):
    d = x.shape[1]
    return pl.pallas_call(
        functools.partial(_sims_body, d=d),
        grid=(NBLK,),
        in_specs=[
            pl.BlockSpec((BLK, d), lambda i: (i, 0)),
            pl.BlockSpec((1, d), lambda i: (0, 0)),
        ],
        out_specs=pl.BlockSpec((1, 1, BLK), lambda i: (i, 0, 0)),
        out_shape=jax.ShapeDtypeStruct((NBLK, 1, BLK), jnp.float32),
    )(x, q).reshape(NBLK, BLK)


def _topk_body(s_ref, vals_ref, idx_ref):
    s = s_ref[...]
    denom = jnp.sum(jnp.exp(s))
    lin = (lax.broadcasted_iota(jnp.int32, (NBLK, BLK), 0) * BLK
           + lax.broadcasted_iota(jnp.int32, (NBLK, BLK), 1))
    lane = lax.broadcasted_iota(jnp.int32, (1, 16), 1)
    vals = jnp.full((1, 16), NEG, jnp.float32)
    idxs = jnp.zeros((1, 16), jnp.int32)
    for k in range(TOP_N):
        m = jnp.max(s)
        sel = s == m
        cand = jnp.min(jnp.where(sel, lin, jnp.int32(2 ** 30)))
        vals = jnp.where(lane == k, m, vals)
        idxs = jnp.where(lane == k, cand, idxs)
        s = jnp.where(lin == cand, NEG, s)
    vals_ref[...] = jnp.exp(vals) / denom
    idx_ref[...] = idxs


def _topk(sims):
    vals, idxs = pl.pallas_call(
        _topk_body,
        out_shape=(jax.ShapeDtypeStruct((1, 16), jnp.float32),
                   jax.ShapeDtypeStruct((1, 16), jnp.int32)),
    )(sims)
    return vals[0, :TOP_N], idxs[0, :TOP_N]


def kernel(gemini_sections, gemini_chapters, gemini_pages,
           voyager_sections, voyager_chapters, voyager_pages,
           gemini_query_embedding, voyager_query_embedding):
    out = []
    for emb, q in ((gemini_sections, gemini_query_embedding),
                   (gemini_chapters, gemini_query_embedding),
                   (gemini_pages, gemini_query_embedding),
                   (voyager_sections, voyager_query_embedding),
                   (voyager_chapters, voyager_query_embedding),
                   (voyager_pages, voyager_query_embedding)):
        vals, idxs = _topk(_sims(emb, q))
        out.append(vals)
        out.append(idxs)
    return tuple(out)


# speedup vs baseline: 4.6403x; 1.1973x over previous
import functools

import jax
import jax.numpy as jnp
from jax import lax
from jax.experimental import pallas as pl

N = 100000
TOP_N = 10
BLK = 2048
NBLK = (N + BLK - 1) // BLK
NPAD = NBLK * BLK
NEG = -1e30


def _sims_body(x_ref, q_ref, o_ref, *, d):
    x = x_ref[...]
    q = q_ref[...]
    dot = jnp.sum(x * q, axis=1)
    sq = jnp.sum(x * x, axis=1)
    qn = jnp.maximum(jnp.sqrt(jnp.sum(q * q)), 1e-8)
    xn = jnp.maximum(jnp.sqrt(sq), 1e-8)
    sims = dot / (xn * qn)
    i = pl.program_id(0)
    row = i * BLK + lax.broadcasted_iota(jnp.int32, (1, BLK), 1)
    o_ref[...] = jnp.where(row < N, sims.reshape(1, BLK), NEG).reshape(1, 1, BLK)


def _sims(x, q):
    d = x.shape[1]
    return pl.pallas_call(
        functools.partial(_sims_body, d=d),
        grid=(NBLK,),
        in_specs=[
            pl.BlockSpec((BLK, d), lambda i: (i, 0)),
            pl.BlockSpec((1, d), lambda i: (0, 0)),
        ],
        out_specs=pl.BlockSpec((1, 1, BLK), lambda i: (i, 0, 0)),
        out_shape=jax.ShapeDtypeStruct((NBLK, 1, BLK), jnp.float32),
    )(x, q).reshape(NBLK, BLK)


def _topk_body(s_ref, vals_ref, idx_ref):
    s = s_ref[...]
    denom = jnp.sum(jnp.exp(s))
    lin = (lax.broadcasted_iota(jnp.int32, (NBLK, BLK), 0) * BLK
           + lax.broadcasted_iota(jnp.int32, (NBLK, BLK), 1))
    lane = lax.broadcasted_iota(jnp.int32, (1, 16), 1)
    vals = jnp.full((1, 16), NEG, jnp.float32)
    idxs = jnp.zeros((1, 16), jnp.int32)
    for k in range(TOP_N):
        m = jnp.max(s)
        sel = s == m
        cand = jnp.min(jnp.where(sel, lin, jnp.int32(2 ** 30)))
        vals = jnp.where(lane == k, m, vals)
        idxs = jnp.where(lane == k, cand, idxs)
        s = jnp.where(lin == cand, NEG, s)
    vals_ref[...] = jnp.exp(vals) / denom
    idx_ref[...] = idxs


def _topk(sims):
    vals, idxs = pl.pallas_call(
        _topk_body,
        out_shape=(jax.ShapeDtypeStruct((1, 16), jnp.float32),
                   jax.ShapeDtypeStruct((1, 16), jnp.int32)),
    )(sims)
    return vals[0, :TOP_N], idxs[0, :TOP_N]


def kernel(gemini_sections, gemini_chapters, gemini_pages,
           voyager_sections, voyager_chapters, voyager_pages,
           gemini_query_embedding, voyager_query_embedding):
    out = []
    for emb, q in ((gemini_sections, gemini_query_embedding),
                   (gemini_chapters, gemini_query_embedding),
                   (gemini_pages, gemini_query_embedding),
                   (voyager_sections, voyager_query_embedding),
                   (voyager_chapters, voyager_query_embedding),
                   (voyager_pages, voyager_query_embedding)):
        vals, idxs = _topk(_sims(emb, q))
        out.append(vals)
        out.append(idxs)
    return tuple(out)


# speedup vs baseline: 4.9885x; 1.0750x over previous
import functools

import jax
import jax.numpy as jnp
from jax import lax
from jax.experimental import pallas as pl

N = 100000
TOP_N = 10
BLK = 4096
NBLK = (N + BLK - 1) // BLK
NPAD = NBLK * BLK
NEG = -1e30


def _sims_body(x_ref, q_ref, o_ref, *, d):
    x = x_ref[...]
    q = q_ref[...]
    dot = jnp.sum(x * q, axis=1)
    sq = jnp.sum(x * x, axis=1)
    qn = jnp.maximum(jnp.sqrt(jnp.sum(q * q)), 1e-8)
    xn = jnp.maximum(jnp.sqrt(sq), 1e-8)
    sims = dot / (xn * qn)
    i = pl.program_id(0)
    row = i * BLK + lax.broadcasted_iota(jnp.int32, (1, BLK), 1)
    o_ref[...] = jnp.where(row < N, sims.reshape(1, BLK), NEG).reshape(1, 1, BLK)


def _sims(x, q):
    d = x.shape[1]
    return pl.pallas_call(
        functools.partial(_sims_body, d=d),
        grid=(NBLK,),
        in_specs=[
            pl.BlockSpec((BLK, d), lambda i: (i, 0)),
            pl.BlockSpec((1, d), lambda i: (0, 0)),
        ],
        out_specs=pl.BlockSpec((1, 1, BLK), lambda i: (i, 0, 0)),
        out_shape=jax.ShapeDtypeStruct((NBLK, 1, BLK), jnp.float32),
    )(x, q).reshape(NBLK, BLK)


def _topk_body(s_ref, vals_ref, idx_ref):
    s = s_ref[...]
    denom = jnp.sum(jnp.exp(s))
    lin = (lax.broadcasted_iota(jnp.int32, (NBLK, BLK), 0) * BLK
           + lax.broadcasted_iota(jnp.int32, (NBLK, BLK), 1))
    lane = lax.broadcasted_iota(jnp.int32, (1, 16), 1)
    vals = jnp.full((1, 16), NEG, jnp.float32)
    idxs = jnp.zeros((1, 16), jnp.int32)
    for k in range(TOP_N):
        m = jnp.max(s)
        sel = s == m
        cand = jnp.min(jnp.where(sel, lin, jnp.int32(2 ** 30)))
        vals = jnp.where(lane == k, m, vals)
        idxs = jnp.where(lane == k, cand, idxs)
        s = jnp.where(lin == cand, NEG, s)
    vals_ref[...] = jnp.exp(vals) / denom
    idx_ref[...] = idxs


def _topk(sims):
    vals, idxs = pl.pallas_call(
        _topk_body,
        out_shape=(jax.ShapeDtypeStruct((1, 16), jnp.float32),
                   jax.ShapeDtypeStruct((1, 16), jnp.int32)),
    )(sims)
    return vals[0, :TOP_N], idxs[0, :TOP_N]


def kernel(gemini_sections, gemini_chapters, gemini_pages,
           voyager_sections, voyager_chapters, voyager_pages,
           gemini_query_embedding, voyager_query_embedding):
    out = []
    for emb, q in ((gemini_sections, gemini_query_embedding),
                   (gemini_chapters, gemini_query_embedding),
                   (gemini_pages, gemini_query_embedding),
                   (voyager_sections, voyager_query_embedding),
                   (voyager_chapters, voyager_query_embedding),
                   (voyager_pages, voyager_query_embedding)):
        vals, idxs = _topk(_sims(emb, q))
        out.append(vals)
        out.append(idxs)
    return tuple(out)
